# baseline (speedup 1.0000x reference)
import functools

import jax
import jax.numpy as jnp
import numpy as np
from jax.experimental import pallas as pl
from jax.experimental.pallas import tpu as pltpu

_N = 10000
_B = 1024
_NB = 10
_NP = _NB * _B
_T = _NB * (_NB + 1) // 2

_IJ = [(i, j) for i in range(_NB) for j in range(i, _NB)]
_I_ARR = np.array([p[0] for p in _IJ] + [_NB - 1], np.int32)
_J_ARR = np.array([p[1] for p in _IJ] + [_NB - 1], np.int32)

_B2 = 2048
_NB2 = 5
_T2 = _NB2 * (_NB2 + 1) // 2
_IJ2 = [(i, j) for i in range(_NB2) for j in range(i, _NB2)]
_I2_ARR = np.array([p[0] for p in _IJ2] + [_NB2 - 1], np.int32)
_J2_ARR = np.array([p[1] for p in _IJ2] + [_NB2 - 1], np.int32)


def _dotT(a, ti):
    return jax.lax.dot_general(a, ti, (((0,), (0,)), ((), ())),
                               preferred_element_type=jnp.float32)


def _mlp(tf_ref, w_ref, b_ref, eps_ref, o_ref, agg_ref, m, last):
    if last:
        pre = (1.0 + eps_ref[0]) * tf_ref[:_N, :] + agg_ref[:_N, :]
        y = jnp.dot(pre.astype(jnp.bfloat16), w_ref[...],
                    preferred_element_type=jnp.float32) + b_ref[...]
        o_ref[...] = jnp.maximum(y, 0.0)
    else:
        pre = (1.0 + eps_ref[0]) * tf_ref[...] + agg_ref[...]
        y = jnp.dot(pre.astype(jnp.bfloat16), w_ref[...],
                    preferred_element_type=jnp.float32) + b_ref[...]
        row = jax.lax.broadcasted_iota(jnp.int32, (_NP, m), 0)
        o_ref[...] = jnp.where(row < _N, jnp.maximum(y, 0.0), 0.0)


def _l0_body(m, *refs):
    (i_ref, j_ref, a_ref, tf_ref, w_ref, b_ref, eps_ref,
     o_ref, agg_ref) = refs
    t = pl.program_id(0)

    @pl.when(t == 0)
    def _():
        agg_ref[...] = jnp.zeros_like(agg_ref)

    @pl.when(t < _T)
    def _():
        i = i_ref[t]
        j = j_ref[t]
        a = a_ref[...].astype(jnp.bfloat16)
        tj = tf_ref[pl.ds(j * _B, _B), :].astype(jnp.bfloat16)
        agg_ref[pl.ds(i * _B, _B), :] += jnp.dot(
            a, tj, preferred_element_type=jnp.float32)

        @pl.when(j != i)
        def _():
            ti = tf_ref[pl.ds(i * _B, _B), :].astype(jnp.bfloat16)
            agg_ref[pl.ds(j * _B, _B), :] += _dotT(a, ti)

    @pl.when(t == _T)
    def _():
        _mlp(tf_ref, w_ref, b_ref, eps_ref, o_ref, agg_ref, m, last=False)


def _l12_body(m, last, *refs):
    (i_ref, j_ref, a_ref, tf_ref, w_ref, b_ref, eps_ref,
     o_ref, agg_ref) = refs
    t = pl.program_id(0)

    @pl.when(t == 0)
    def _():
        agg_ref[...] = jnp.zeros_like(agg_ref)

    @pl.when(t < _T2)
    def _():
        i = i_ref[t]
        j = j_ref[t]

        @pl.when(j != i)
        def _():
            a = a_ref[...].astype(jnp.bfloat16)
            tj = tf_ref[pl.ds(j * _B2, _B2), :].astype(jnp.bfloat16)
            agg_ref[pl.ds(i * _B2, _B2), :] += jnp.dot(
                a, tj, preferred_element_type=jnp.float32)
            ti = tf_ref[pl.ds(i * _B2, _B2), :].astype(jnp.bfloat16)
            agg_ref[pl.ds(j * _B2, _B2), :] += _dotT(a, ti)

        @pl.when(j == i)
        def _():
            base = i * _B2
            a00 = a_ref[:_B, :_B].astype(jnp.bfloat16)
            a01 = a_ref[:_B, _B:].astype(jnp.bfloat16)
            a11 = a_ref[_B:, _B:].astype(jnp.bfloat16)
            t0 = tf_ref[pl.ds(base, _B), :].astype(jnp.bfloat16)
            t1 = tf_ref[pl.ds(base + _B, _B), :].astype(jnp.bfloat16)
            agg_ref[pl.ds(base, _B), :] += (
                jnp.dot(a00, t0, preferred_element_type=jnp.float32)
                + jnp.dot(a01, t1, preferred_element_type=jnp.float32))
            agg_ref[pl.ds(base + _B, _B), :] += (
                jnp.dot(a11, t1, preferred_element_type=jnp.float32)
                + _dotT(a01, t0))

    @pl.when(t == _T2)
    def _():
        _mlp(tf_ref, w_ref, b_ref, eps_ref, o_ref, agg_ref, m, last)


def _layer0(A, t_in, W, b, eps_i):
    k = t_in.shape[1]
    m = W.shape[1]
    grid_spec = pltpu.PrefetchScalarGridSpec(
        num_scalar_prefetch=2,
        grid=(_T + 1,),
        in_specs=[
            pl.BlockSpec((_B, _B), lambda t, i_a, j_a: (i_a[t], j_a[t])),
            pl.BlockSpec((_NP, k), lambda t, i_a, j_a: (0, 0)),
            pl.BlockSpec((k, m), lambda t, i_a, j_a: (0, 0)),
            pl.BlockSpec((1, m), lambda t, i_a, j_a: (0, 0)),
            pl.BlockSpec(memory_space=pltpu.SMEM),
        ],
        out_specs=pl.BlockSpec((_NP, m), lambda t, i_a, j_a: (0, 0)),
        scratch_shapes=[pltpu.VMEM((_NP, k), jnp.float32)],
    )
    return pl.pallas_call(
        functools.partial(_l0_body, m),
        grid_spec=grid_spec,
        out_shape=jax.ShapeDtypeStruct((_NP, m), jnp.float32),
        compiler_params=pltpu.CompilerParams(
            dimension_semantics=("arbitrary",)),
    )(jnp.asarray(_I_ARR), jnp.asarray(_J_ARR), A, t_in,
      W.astype(jnp.bfloat16), b.reshape(1, m), eps_i.reshape(1,))


def _layer12(cache, t_in, W, b, eps_i, last):
    k = t_in.shape[1]
    m = W.shape[1]
    n_out = _N if last else _NP
    grid_spec = pltpu.PrefetchScalarGridSpec(
        num_scalar_prefetch=2,
        grid=(_T2 + 1,),
        in_specs=[
            pl.BlockSpec((_B2, _B2), lambda t, i_a, j_a: (i_a[t], j_a[t])),
            pl.BlockSpec((_NP, k), lambda t, i_a, j_a: (0, 0)),
            pl.BlockSpec((k, m), lambda t, i_a, j_a: (0, 0)),
            pl.BlockSpec((1, m), lambda t, i_a, j_a: (0, 0)),
            pl.BlockSpec(memory_space=pltpu.SMEM),
        ],
        out_specs=pl.BlockSpec((n_out, m), lambda t, i_a, j_a: (0, 0)),
        scratch_shapes=[pltpu.VMEM((_NP, k), jnp.float32)],
    )
    return pl.pallas_call(
        functools.partial(_l12_body, m, last),
        grid_spec=grid_spec,
        out_shape=jax.ShapeDtypeStruct((n_out, m), jnp.float32),
        compiler_params=pltpu.CompilerParams(
            dimension_semantics=("arbitrary",)),
    )(jnp.asarray(_I2_ARR), jnp.asarray(_J2_ARR), cache, t_in,
      W.astype(jnp.bfloat16), b.reshape(1, m), eps_i.reshape(1,))


def kernel(A, X, epsilon_dim, h, W0, b0, W1, b1, W2, b2, eps):
    n = X.shape[0]
    eps_dim = W0.shape[0] - X.shape[1] - h.shape[1]
    bern = jax.random.bernoulli(jax.random.key(42), 0.5, (n, eps_dim)).astype(jnp.float32)
    t0 = jnp.concatenate([X, bern, h], axis=1)
    t0 = jnp.pad(t0, ((0, _NP - _N), (0, 0)))
    t1 = _layer0(A, t0, W0, b0, eps[0])
    t2 = _layer12(A, t1, W1, b1, eps[1], last=False)
    return _layer12(A, t2, W2, b2, eps[2], last=True)


# speedup vs baseline: 1.0267x; 1.0267x over previous
import functools

import jax
import jax.numpy as jnp
import numpy as np
from jax.experimental import pallas as pl
from jax.experimental.pallas import tpu as pltpu

_N = 10000
_B = 1024
_NB = 10
_NP = _NB * _B
_T = _NB * (_NB + 1) // 2

_IJ = [(i, j) for i in range(_NB) for j in range(i, _NB)]
_I_ARR = np.array([p[0] for p in _IJ] + [_NB - 1], np.int32)
_J_ARR = np.array([p[1] for p in _IJ] + [_NB - 1], np.int32)

_B2 = 2048
_NB2 = 5
_T2 = _NB2 * (_NB2 + 1) // 2
_IJ2 = [(i, j) for i in range(_NB2) for j in range(i, _NB2)]
_I2_ARR = np.array([p[0] for p in _IJ2] + [_NB2 - 1], np.int32)
_J2_ARR = np.array([p[1] for p in _IJ2] + [_NB2 - 1], np.int32)


def _dotT(a, ti):
    return jax.lax.dot_general(a, ti, (((0,), (0,)), ((), ())),
                               preferred_element_type=jnp.float32)


def _mlp(tf_ref, w_ref, b_ref, eps_ref, o_ref, agg_ref, m, last):
    if last:
        pre = (1.0 + eps_ref[0]) * tf_ref[:_N, :] + agg_ref[:_N, :]
        y = jnp.dot(pre.astype(jnp.bfloat16), w_ref[...],
                    preferred_element_type=jnp.float32) + b_ref[...]
        o_ref[...] = jnp.maximum(y, 0.0)
    else:
        pre = (1.0 + eps_ref[0]) * tf_ref[...] + agg_ref[...]
        y = jnp.dot(pre.astype(jnp.bfloat16), w_ref[...],
                    preferred_element_type=jnp.float32) + b_ref[...]
        row = jax.lax.broadcasted_iota(jnp.int32, (_NP, m), 0)
        o_ref[...] = jnp.where(row < _N, jnp.maximum(y, 0.0), 0.0)


def _l0_body(m, *refs):
    (i_ref, j_ref, a_ref, x_ref, bern_ref, h_ref, w_ref, b_ref, eps_ref,
     o_ref, cache_ref, tf_ref, agg_ref) = refs
    t = pl.program_id(0)

    @pl.when(t == 0)
    def _():
        agg_ref[...] = jnp.zeros_like(agg_ref)
        if _NP > _N:
            tf_ref[_N:, :] = jnp.zeros_like(tf_ref[_N:, :])
        tf_ref[:_N, :128] = x_ref[...]
        tf_ref[:_N, 128:144] = bern_ref[...]
        tf_ref[:_N, 144:160] = h_ref[...]

    @pl.when(t < _T)
    def _():
        i = i_ref[t]
        j = j_ref[t]
        a = a_ref[...].astype(jnp.bfloat16)
        cache_ref[...] = a
        tj = tf_ref[pl.ds(j * _B, _B), :].astype(jnp.bfloat16)
        agg_ref[pl.ds(i * _B, _B), :] += jnp.dot(
            a, tj, preferred_element_type=jnp.float32)

        @pl.when(j != i)
        def _():
            ti = tf_ref[pl.ds(i * _B, _B), :].astype(jnp.bfloat16)
            agg_ref[pl.ds(j * _B, _B), :] += _dotT(a, ti)

    @pl.when(t == _T)
    def _():
        _mlp(tf_ref, w_ref, b_ref, eps_ref, o_ref, agg_ref, m, last=False)


def _l12_body(m, last, *refs):
    (i_ref, j_ref, a_ref, tf_ref, w_ref, b_ref, eps_ref,
     o_ref, agg_ref) = refs
    t = pl.program_id(0)

    @pl.when(t == 0)
    def _():
        agg_ref[...] = jnp.zeros_like(agg_ref)

    @pl.when(t < _T2)
    def _():
        i = i_ref[t]
        j = j_ref[t]

        @pl.when(j != i)
        def _():
            a = a_ref[...]
            tj = tf_ref[pl.ds(j * _B2, _B2), :].astype(jnp.bfloat16)
            agg_ref[pl.ds(i * _B2, _B2), :] += jnp.dot(
                a, tj, preferred_element_type=jnp.float32)
            ti = tf_ref[pl.ds(i * _B2, _B2), :].astype(jnp.bfloat16)
            agg_ref[pl.ds(j * _B2, _B2), :] += _dotT(a, ti)

        @pl.when(j == i)
        def _():
            base = i * _B2
            a00 = a_ref[:_B, :_B]
            a01 = a_ref[:_B, _B:]
            a11 = a_ref[_B:, _B:]
            t0 = tf_ref[pl.ds(base, _B), :].astype(jnp.bfloat16)
            t1 = tf_ref[pl.ds(base + _B, _B), :].astype(jnp.bfloat16)
            agg_ref[pl.ds(base, _B), :] += (
                jnp.dot(a00, t0, preferred_element_type=jnp.float32)
                + jnp.dot(a01, t1, preferred_element_type=jnp.float32))
            agg_ref[pl.ds(base + _B, _B), :] += (
                jnp.dot(a11, t1, preferred_element_type=jnp.float32)
                + _dotT(a01, t0))

    @pl.when(t == _T2)
    def _():
        _mlp(tf_ref, w_ref, b_ref, eps_ref, o_ref, agg_ref, m, last)


def _layer0(A, X, bern, h, W, b, eps_i):
    k = X.shape[1] + bern.shape[1] + h.shape[1]
    m = W.shape[1]
    grid_spec = pltpu.PrefetchScalarGridSpec(
        num_scalar_prefetch=2,
        grid=(_T + 1,),
        in_specs=[
            pl.BlockSpec((_B, _B), lambda t, i_a, j_a: (i_a[t], j_a[t])),
            pl.BlockSpec((_N, 128), lambda t, i_a, j_a: (0, 0)),
            pl.BlockSpec((_N, 16), lambda t, i_a, j_a: (0, 0)),
            pl.BlockSpec((_N, 16), lambda t, i_a, j_a: (0, 0)),
            pl.BlockSpec((k, m), lambda t, i_a, j_a: (0, 0)),
            pl.BlockSpec((1, m), lambda t, i_a, j_a: (0, 0)),
            pl.BlockSpec(memory_space=pltpu.SMEM),
        ],
        out_specs=[
            pl.BlockSpec((_NP, m), lambda t, i_a, j_a: (0, 0)),
            pl.BlockSpec((_B, _B), lambda t, i_a, j_a: (i_a[t], j_a[t])),
        ],
        scratch_shapes=[pltpu.VMEM((_NP, k), jnp.float32),
                        pltpu.VMEM((_NP, k), jnp.float32)],
    )
    return pl.pallas_call(
        functools.partial(_l0_body, m),
        grid_spec=grid_spec,
        out_shape=[
            jax.ShapeDtypeStruct((_NP, m), jnp.float32),
            jax.ShapeDtypeStruct((_NP, _NP), jnp.bfloat16),
        ],
        compiler_params=pltpu.CompilerParams(
            dimension_semantics=("arbitrary",)),
    )(jnp.asarray(_I_ARR), jnp.asarray(_J_ARR), A, X, bern, h,
      W.astype(jnp.bfloat16), b.reshape(1, m), eps_i.reshape(1,))


def _layer12(cache, t_in, W, b, eps_i, last):
    k = t_in.shape[1]
    m = W.shape[1]
    n_out = _N if last else _NP
    grid_spec = pltpu.PrefetchScalarGridSpec(
        num_scalar_prefetch=2,
        grid=(_T2 + 1,),
        in_specs=[
            pl.BlockSpec((_B2, _B2), lambda t, i_a, j_a: (i_a[t], j_a[t])),
            pl.BlockSpec((_NP, k), lambda t, i_a, j_a: (0, 0)),
            pl.BlockSpec((k, m), lambda t, i_a, j_a: (0, 0)),
            pl.BlockSpec((1, m), lambda t, i_a, j_a: (0, 0)),
            pl.BlockSpec(memory_space=pltpu.SMEM),
        ],
        out_specs=pl.BlockSpec((n_out, m), lambda t, i_a, j_a: (0, 0)),
        scratch_shapes=[pltpu.VMEM((_NP, k), jnp.float32)],
    )
    return pl.pallas_call(
        functools.partial(_l12_body, m, last),
        grid_spec=grid_spec,
        out_shape=jax.ShapeDtypeStruct((n_out, m), jnp.float32),
        compiler_params=pltpu.CompilerParams(
            dimension_semantics=("arbitrary",)),
    )(jnp.asarray(_I2_ARR), jnp.asarray(_J2_ARR), cache, t_in,
      W.astype(jnp.bfloat16), b.reshape(1, m), eps_i.reshape(1,))


def kernel(A, X, epsilon_dim, h, W0, b0, W1, b1, W2, b2, eps):
    n = X.shape[0]
    eps_dim = W0.shape[0] - X.shape[1] - h.shape[1]
    bern = jax.random.bernoulli(
        jax.random.key(42), 0.5, (n, eps_dim)).astype(jnp.float32)
    t1, cache = _layer0(A, X, bern, h, W0, b0, eps[0])
    t2 = _layer12(cache, t1, W1, b1, eps[1], last=False)
    return _layer12(cache, t2, W2, b2, eps[2], last=True)
